# Initial kernel scaffold; baseline (speedup 1.0000x reference)
import jax
import jax.numpy as jnp
from jax.experimental import pallas as pl


def kernel(melspecs, centroids):
    raise NotImplementedError("write your pallas kernel here")


# speedup vs baseline: 190.2105x; 190.2105x over previous
import functools

import jax
import jax.numpy as jnp
from jax import lax
from jax.experimental import pallas as pl
from jax.experimental.pallas import tpu as pltpu
from jax.experimental.pallas import tpu_sc as plsc

B, T, M, K = 32, 2048, 80, 8
L = 16
PHASES = M // L
TOTAL = B * T * M
NW = 32
ELEMS_PER_W = TOTAL // NW
CHUNK_ROWS = 256
CHUNK = CHUNK_ROWS * M
NCHUNKS = ELEMS_PER_W // CHUNK


def _discretize(x_hbm, c_hbm, out_hbm, xbuf, obuf, cbuf):
    nc = 2
    wid = lax.axis_index("s") * nc + lax.axis_index("c")
    base = wid * ELEMS_PER_W

    pltpu.sync_copy(c_hbm, cbuf)
    cv = [[cbuf[j * K + k, :] for k in range(K)] for j in range(PHASES)]

    def chunk_body(g, _):
        off = base + g * CHUNK
        pltpu.sync_copy(x_hbm.at[pl.ds(off, CHUNK)], xbuf)

        def row_body(r, _):
            ro = r * M
            for j in range(PHASES):
                x = xbuf[pl.ds(ro + j * L, L)]
                c = cv[j]
                best_v = c[0]
                best_d = jnp.abs(x - c[0])
                for k in range(1, K):
                    d = jnp.abs(x - c[k])
                    take = d < best_d
                    best_v = jnp.where(take, c[k], best_v)
                    best_d = jnp.minimum(d, best_d)
                obuf[pl.ds(ro + j * L, L)] = best_v
            return 0

        lax.fori_loop(0, CHUNK_ROWS, row_body, 0)
        pltpu.sync_copy(obuf, out_hbm.at[pl.ds(off, CHUNK)])
        return 0

    lax.fori_loop(0, NCHUNKS, chunk_body, 0)


@jax.jit
def kernel(melspecs, centroids):
    ctab = jnp.transpose(centroids.reshape(PHASES, L, K), (0, 2, 1))
    ctab = ctab.reshape(PHASES * K, L)
    x = melspecs.reshape(TOTAL)

    mesh = plsc.VectorSubcoreMesh(
        core_axis_name="c", subcore_axis_name="s", num_cores=2, num_subcores=16
    )
    out = pl.kernel(
        _discretize,
        out_type=jax.ShapeDtypeStruct((TOTAL,), jnp.float32),
        mesh=mesh,
        scratch_types=[
            pltpu.VMEM((CHUNK,), jnp.float32),
            pltpu.VMEM((CHUNK,), jnp.float32),
            pltpu.VMEM((PHASES * K, L), jnp.float32),
        ],
    )(x, ctab)
    return out.reshape(B, T, M)
